# Initial kernel scaffold; baseline (speedup 1.0000x reference)
import jax
import jax.numpy as jnp
from jax.experimental import pallas as pl


def kernel(x, emb, W, b):
    raise NotImplementedError("write your pallas kernel here")


# speedup vs baseline: 10.7496x; 10.7496x over previous
import functools

import jax
import jax.numpy as jnp
from jax import lax
from jax.experimental import pallas as pl
from jax.experimental.pallas import tpu as pltpu
from jax.experimental.pallas import tpu_sc as plsc

B = 4096
L = 50
D = 128
NC = 2
NS = 16
NW = NC * NS
PB = B // NW
GE = 2
CR = GE * L
NCH = PB // GE
XROWS = B * L // CR
XPW = XROWS // NW


def _sc_body(xr_h, emb_h, acc_h, xv, rowbuf, accbuf, sem0, sem1):
    cid = lax.axis_index("c")
    sid = lax.axis_index("s")
    wid = sid * NC + cid

    pltpu.sync_copy(xr_h.at[pl.ds(wid * XPW, XPW)], xv)

    sems = (sem0, sem1)

    pltpu.async_copy(emb_h.at[xv.at[0]], rowbuf.at[0], sem0)
    pltpu.async_copy(emb_h.at[xv.at[1]], rowbuf.at[1], sem1)

    def outer(j2, carry):
        for bslot in range(2):
            jj = j2 * 2 + bslot
            pltpu.make_async_copy(
                emb_h.at[xv.at[jj]], rowbuf.at[bslot], sems[bslot]).wait()
            for e in range(GE):
                bi = jj * GE + e

                def rbody(r, acc, _e=e, _bslot=bslot):
                    row = _e * L + r
                    return tuple(
                        acc[d] + rowbuf[_bslot, row, pl.ds(16 * d, 16)]
                        for d in range(8))

                acc = lax.fori_loop(
                    0, L, rbody,
                    tuple(jnp.zeros((16,), jnp.float32) for _ in range(8)))
                for d in range(8):
                    accbuf[bi, pl.ds(16 * d, 16)] = acc[d]

            @pl.when(jj + 2 < NCH)
            def _(_bslot=bslot, _jj=jj):
                pltpu.async_copy(
                    emb_h.at[xv.at[_jj + 2]], rowbuf.at[_bslot], sems[_bslot])
        return carry

    lax.fori_loop(0, NCH // 2, outer, 0)

    pltpu.sync_copy(accbuf, acc_h.at[pl.ds(wid * PB, PB)])


def _tc_body(acc_ref, w_ref, b_ref, o_ref):
    m = acc_ref[...] * (1.0 / L)
    s = 1.0 / (1.0 + jnp.exp(-m))
    t = jnp.sum(s * w_ref[...], axis=1, keepdims=True) + b_ref[0, 0]
    o_ref[...] = 1.0 / (1.0 + jnp.exp(-t))


@jax.jit
def _run(xr, emb, w2d, b2d):
    mesh = plsc.VectorSubcoreMesh(core_axis_name="c", subcore_axis_name="s",
                                  num_cores=NC, num_subcores=NS)
    sc = pl.kernel(
        _sc_body,
        out_type=jax.ShapeDtypeStruct((B, D), jnp.float32),
        mesh=mesh,
        scratch_types=[
            pltpu.VMEM((XPW, CR), jnp.int32),
            pltpu.VMEM((2, CR, D), jnp.float32),
            pltpu.VMEM((PB, D), jnp.float32),
            pltpu.SemaphoreType.DMA,
            pltpu.SemaphoreType.DMA,
        ],
    )
    acc = sc(xr, emb)
    out = pl.pallas_call(
        _tc_body,
        out_shape=jax.ShapeDtypeStruct((B, 1), jnp.float32),
    )(acc, w2d, b2d)
    return out


def kernel(x, emb, W, b):
    xr = x.reshape(XROWS, CR).astype(jnp.int32)
    w2d = W.reshape(1, D).astype(jnp.float32)
    b2d = b.reshape(1, 1).astype(jnp.float32)
    return _run(xr, emb, w2d, b2d)
